# baseline (speedup 1.0000x reference)
import functools

import jax
import jax.numpy as jnp
from jax import lax
from jax.experimental import pallas as pl
from jax.experimental.pallas import tpu as pltpu
from jax.experimental.pallas import tpu_sc as plsc

N = 10000
D = 128
G = 64
E = 320000
NC, NS = 2, 16
NW = NC * NS
CH = 128
K = 80
KAL = 88
GRP = 2
EPAD = NW * K * CH
NP = N + 112
RPS = NP // NS
DEGW = 16

_f32 = jnp.float32


def _sc_mesh():
    return plsc.VectorSubcoreMesh(core_axis_name="c", subcore_axis_name="s")


def _sc_degree(dst_idx, ones_rows, zrows):
    @functools.partial(
        pl.kernel,
        out_type=jax.ShapeDtypeStruct((NC, NP, D), _f32),
        mesh=_sc_mesh(),
        scratch_types=[
            pltpu.VMEM((KAL, CH), jnp.int32),
            pltpu.VMEM((CH, D), _f32),
            pltpu.VMEM_SHARED((NP, D), _f32),
            pltpu.SemaphoreType.DMA,
        ],
    )
    def k(dst_hbm, ones_hbm, z_hbm, out_hbm, dst_v, ones_v, accd, sem):
        c = lax.axis_index("c")
        s = lax.axis_index("s")
        w = c * NS + s
        pltpu.sync_copy(z_hbm.at[pl.ds(s * RPS, RPS)], accd.at[pl.ds(s * RPS, RPS)])
        pltpu.sync_copy(dst_hbm.at[w], dst_v)
        pltpu.sync_copy(ones_hbm, ones_v)
        plsc.subcore_barrier()

        def body(t, carry):
            j = t * 8
            for b in range(8):
                pltpu.async_copy(ones_v, accd.at[dst_v.at[j + b]], sem, add=True)
            for b in range(8):
                pltpu.make_async_copy(ones_v, accd.at[dst_v.at[j + b]], sem).wait()
            return carry

        lax.fori_loop(0, K // 8, body, 0)
        plsc.subcore_barrier()
        pltpu.sync_copy(accd.at[pl.ds(s * RPS, RPS)],
                        out_hbm.at[c, pl.ds(s * RPS, RPS)])

    return k(dst_idx, ones_rows, zrows)


def _sc_scatter(y, packed_idx, zrows):
    @functools.partial(
        pl.kernel,
        out_type=jax.ShapeDtypeStruct((NC, NP, D), _f32),
        mesh=_sc_mesh(),
        scratch_types=[
            pltpu.VMEM((KAL, CH), jnp.int32),
            pltpu.VMEM((2, CH), jnp.int32),
            pltpu.VMEM((2, CH), jnp.int32),
            pltpu.VMEM((2, CH, D), _f32),
            pltpu.VMEM_SHARED((NP, D), _f32),
            pltpu.SemaphoreType.DMA,
            pltpu.SemaphoreType.DMA,
            pltpu.SemaphoreType.DMA,
            pltpu.SemaphoreType.DMA,
        ],
    )
    def k(y_hbm, pk_hbm, z_hbm, out_hbm,
          pk_v, src_r, dst_r, rows, acc, gs0, gs1, ss0, ss1):
        c = lax.axis_index("c")
        s = lax.axis_index("s")
        w = c * NS + s
        pltpu.sync_copy(z_hbm.at[pl.ds(s * RPS, RPS)], acc.at[pl.ds(s * RPS, RPS)])
        pltpu.sync_copy(pk_hbm.at[w], pk_v)
        plsc.subcore_barrier()
        gsem = (gs0, gs1)
        ssem = (ss0, ss1)

        def unpack_src(j, p):
            for i in range(CH // 16):
                pv = pk_v[j, pl.ds(i * 16, 16)]
                src_r[p, pl.ds(i * 16, 16)] = lax.bitwise_and(pv, 0xFFFF)

        def unpack_dst(j, p):
            for i in range(CH // 16):
                pv = pk_v[j, pl.ds(i * 16, 16)]
                dst_r[p, pl.ds(i * 16, 16)] = lax.shift_right_logical(pv, 16)

        def fire_gather(j, p):
            pltpu.async_copy(y_hbm.at[src_r.at[p]], rows.at[p], gsem[p])

        def wait_gather(p):
            pltpu.make_async_copy(y_hbm.at[src_r.at[p]], rows.at[p],
                                  gsem[p]).wait()

        def fire_scatter(p):
            pltpu.async_copy(rows.at[p], acc.at[dst_r.at[p]], ssem[p], add=True)

        def wait_scatter(p):
            pltpu.make_async_copy(rows.at[p], acc.at[dst_r.at[p]],
                                  ssem[p]).wait()

        def phase(j, p):
            wait_gather(p)
            unpack_dst(j, p)
            fire_scatter(p)
            wait_scatter(p)
            unpack_src(j + 2, p)
            fire_gather(j + 2, p)

        unpack_src(0, 0)
        unpack_src(1, 1)
        fire_gather(0, 0)
        fire_gather(1, 1)

        def body(t, carry):
            phase(2 * t, 0)
            phase(2 * t + 1, 1)
            return carry

        lax.fori_loop(0, K // 2, body, 0)
        wait_gather(0)
        wait_gather(1)
        plsc.subcore_barrier()
        pltpu.sync_copy(acc.at[pl.ds(s * RPS, RPS)],
                        out_hbm.at[c, pl.ds(s * RPS, RPS)])

    return k(y, packed_idx, zrows)


def _tc_stage1(degp, x, W1):
    def body(degp_ref, x_ref, w_ref, dinv_ref, y_ref):
        dp = degp_ref[...]
        deg = dp[0, :N, :1] + dp[1, :N, :1] + 1.0
        dinv = lax.rsqrt(deg)
        dinv_ref[...] = dinv
        xw = jnp.dot(x_ref[...], w_ref[...], preferred_element_type=_f32)
        y_ref[...] = xw * dinv

    return pl.pallas_call(
        body,
        out_shape=(jax.ShapeDtypeStruct((N, 1), _f32),
                   jax.ShapeDtypeStruct((N, D), _f32)),
    )(degp, x, W1)


def _bn_relu(accp_ref, y_ref, dinv_ref, b_ref, g_ref, be_ref):
    ap = accp_ref[...]
    acc = ap[0, :N, :] + ap[1, :N, :] + y_ref[...]
    r = jnp.maximum(acc * dinv_ref[...] + b_ref[...], 0.0)
    m = jnp.mean(r, axis=0, keepdims=True)
    v = jnp.mean(r * r, axis=0, keepdims=True) - m * m
    return (r - m) * lax.rsqrt(v + 1e-5) * g_ref[...] + be_ref[...]


def _tc_layer(accp, y, dinv, b, g, be, Wn):
    def body(accp_ref, y_ref, dinv_ref, b_ref, g_ref, be_ref, w_ref, yn_ref):
        h = _bn_relu(accp_ref, y_ref, dinv_ref, b_ref, g_ref, be_ref)
        yn_ref[...] = jnp.dot(h, w_ref[...], preferred_element_type=_f32) * dinv_ref[...]

    return pl.pallas_call(
        body,
        out_shape=jax.ShapeDtypeStruct((N, D), _f32),
    )(accp, y, dinv, b, g, be, Wn)


def _tc_final(accp, y, dinv, b, g, be, batch2d, lw1, lb1, lw2, lb2):
    def body(accp_ref, y_ref, dinv_ref, b_ref, g_ref, be_ref, bat_ref,
             lw1_ref, lb1_ref, lw2_ref, lb2_ref, out_ref, pooled_ref):
        h = _bn_relu(accp_ref, y_ref, dinv_ref, b_ref, g_ref, be_ref)
        bat = bat_ref[...]
        neg = _f32(-jnp.inf)

        def pool_one(gi, carry):
            row = jnp.max(jnp.where(bat == gi, h, neg), axis=0, keepdims=True)
            pooled_ref[pl.ds(gi, 1), :] = row
            return carry

        lax.fori_loop(0, G, pool_one, 0)
        pooled = pooled_ref[...]
        h2 = jnp.maximum(
            jnp.dot(pooled, lw1_ref[...], preferred_element_type=_f32) + lb1_ref[...],
            0.0)
        out_ref[...] = jnp.dot(h2, lw2_ref[...], preferred_element_type=_f32) + lb2_ref[...]

    return pl.pallas_call(
        body,
        out_shape=jax.ShapeDtypeStruct((G, 1), _f32),
        scratch_shapes=[pltpu.VMEM((G, D), _f32)],
    )(accp, y, dinv, b, g, be, batch2d, lw1, lb1, lw2, lb2)


def kernel(x, edge_attr, edge_index, batch,
           W1, b1, g1, be1, W2, b2, g2, be2, W3, b3, g3, be3,
           lw1, lb1, lw2, lb2):
    pad = EPAD - E
    srcp = jnp.concatenate(
        [edge_index[0], jnp.zeros((pad,), jnp.int32)]).reshape(NW, K, CH)
    dstp = jnp.concatenate(
        [edge_index[1], jnp.full((pad,), N, jnp.int32)]).reshape(NW, K, CH)
    srcp = jnp.concatenate(
        [srcp, jnp.zeros((NW, KAL - K, CH), jnp.int32)], axis=1)
    dstp = jnp.concatenate(
        [dstp, jnp.full((NW, KAL - K, CH), N, jnp.int32)], axis=1)
    packed = jnp.bitwise_or(srcp, jnp.left_shift(dstp, 16))
    zrows = jnp.zeros((NP, D), _f32)
    batch2d = batch.reshape(N, 1)
    b1r, g1r, be1r = b1.reshape(1, D), g1.reshape(1, D), be1.reshape(1, D)
    b2r, g2r, be2r = b2.reshape(1, D), g2.reshape(1, D), be2.reshape(1, D)
    b3r, g3r, be3r = b3.reshape(1, D), g3.reshape(1, D), be3.reshape(1, D)
    lb1r = lb1.reshape(1, D // 2)
    lb2r = lb2.reshape(1, 1)

    degp = _sc_degree(dstp, jnp.ones((CH, D), _f32), zrows)
    dinv, y1 = _tc_stage1(degp, x, W1)
    acc1 = _sc_scatter(y1, packed, zrows)
    y2 = _tc_layer(acc1, y1, dinv, b1r, g1r, be1r, W2)
    acc2 = _sc_scatter(y2, packed, zrows)
    y3 = _tc_layer(acc2, y2, dinv, b2r, g2r, be2r, W3)
    acc3 = _sc_scatter(y3, packed, zrows)
    return _tc_final(acc3, y3, dinv, b3r, g3r, be3r, batch2d,
                     lw1, lb1r, lw2, lb2r)


# speedup vs baseline: 1.2946x; 1.2946x over previous
import functools

import jax
import jax.numpy as jnp
from jax import lax
from jax.experimental import pallas as pl
from jax.experimental.pallas import tpu as pltpu
from jax.experimental.pallas import tpu_sc as plsc

N = 10000
D = 128
G = 64
E = 320000
NC, NS = 2, 16
NW = NC * NS
CH = 128
K = 80
KAL = 88
GRP = 2
EPAD = NW * K * CH
NP = N + 112
RPS = NP // NS
DEGW = 16

_f32 = jnp.float32


def _sc_mesh():
    return plsc.VectorSubcoreMesh(core_axis_name="c", subcore_axis_name="s")


def _sc_degree(dst_idx, ones_rows, zrows):
    @functools.partial(
        pl.kernel,
        out_type=jax.ShapeDtypeStruct((NC, NP, D), _f32),
        mesh=_sc_mesh(),
        scratch_types=[
            pltpu.VMEM((KAL, CH), jnp.int32),
            pltpu.VMEM((CH, D), _f32),
            pltpu.VMEM_SHARED((NP, D), _f32),
            pltpu.SemaphoreType.DMA,
        ],
    )
    def k(dst_hbm, ones_hbm, z_hbm, out_hbm, dst_v, ones_v, accd, sem):
        c = lax.axis_index("c")
        s = lax.axis_index("s")
        w = c * NS + s
        pltpu.sync_copy(z_hbm.at[pl.ds(s * RPS, RPS)], accd.at[pl.ds(s * RPS, RPS)])
        pltpu.sync_copy(dst_hbm.at[w], dst_v)
        pltpu.sync_copy(ones_hbm, ones_v)
        plsc.subcore_barrier()

        def body(t, carry):
            j = t * 8
            for b in range(8):
                pltpu.async_copy(ones_v, accd.at[dst_v.at[j + b]], sem, add=True)
            for b in range(8):
                pltpu.make_async_copy(ones_v, accd.at[dst_v.at[j + b]], sem).wait()
            return carry

        lax.fori_loop(0, K // 8, body, 0)
        plsc.subcore_barrier()
        pltpu.sync_copy(accd.at[pl.ds(s * RPS, RPS)],
                        out_hbm.at[c, pl.ds(s * RPS, RPS)])

    return k(dst_idx, ones_rows, zrows)


def _sc_scatter(y, src_idx, dst_idx, zrows):
    @functools.partial(
        pl.kernel,
        out_type=jax.ShapeDtypeStruct((NC, NP, D), _f32),
        mesh=_sc_mesh(),
        scratch_types=[
            pltpu.VMEM((KAL, CH), jnp.int32),
            pltpu.VMEM((KAL, CH), jnp.int32),
            pltpu.VMEM((CH, D), _f32),
            pltpu.VMEM_SHARED((NP, D), _f32),
            pltpu.SemaphoreType.DMA,
        ],
    )
    def k(y_hbm, src_hbm, dst_hbm, z_hbm, out_hbm,
          src_v, dst_v, rows_v, acc, sem):
        c = lax.axis_index("c")
        s = lax.axis_index("s")
        w = c * NS + s
        pltpu.sync_copy(z_hbm.at[pl.ds(s * RPS, RPS)], acc.at[pl.ds(s * RPS, RPS)])
        pltpu.sync_copy(src_hbm.at[w], src_v)
        pltpu.sync_copy(dst_hbm.at[w], dst_v)
        plsc.subcore_barrier()

        def body(j, carry):
            pltpu.async_copy(y_hbm.at[src_v.at[j]], rows_v, sem).wait()
            pltpu.sync_copy(rows_v, acc.at[dst_v.at[j]], add=True)
            return carry

        lax.fori_loop(0, K, body, 0)
        plsc.subcore_barrier()
        pltpu.sync_copy(acc.at[pl.ds(s * RPS, RPS)],
                        out_hbm.at[c, pl.ds(s * RPS, RPS)])

    return k(y, src_idx, dst_idx, zrows)


def _tc_stage1(degp, x, W1):
    def body(degp_ref, x_ref, w_ref, dinv_ref, y_ref):
        dp = degp_ref[...]
        deg = dp[0, :N, :1] + dp[1, :N, :1] + 1.0
        dinv = lax.rsqrt(deg)
        dinv_ref[...] = dinv
        xw = jnp.dot(x_ref[...], w_ref[...], preferred_element_type=_f32)
        y_ref[...] = xw * dinv

    return pl.pallas_call(
        body,
        out_shape=(jax.ShapeDtypeStruct((N, 1), _f32),
                   jax.ShapeDtypeStruct((N, D), _f32)),
    )(degp, x, W1)


def _bn_relu(accp_ref, y_ref, dinv_ref, b_ref, g_ref, be_ref):
    ap = accp_ref[...]
    acc = ap[0, :N, :] + ap[1, :N, :] + y_ref[...]
    r = jnp.maximum(acc * dinv_ref[...] + b_ref[...], 0.0)
    m = jnp.mean(r, axis=0, keepdims=True)
    v = jnp.mean(r * r, axis=0, keepdims=True) - m * m
    return (r - m) * lax.rsqrt(v + 1e-5) * g_ref[...] + be_ref[...]


def _tc_layer(accp, y, dinv, b, g, be, Wn):
    def body(accp_ref, y_ref, dinv_ref, b_ref, g_ref, be_ref, w_ref, yn_ref):
        h = _bn_relu(accp_ref, y_ref, dinv_ref, b_ref, g_ref, be_ref)
        yn_ref[...] = jnp.dot(h, w_ref[...], preferred_element_type=_f32) * dinv_ref[...]

    return pl.pallas_call(
        body,
        out_shape=jax.ShapeDtypeStruct((N, D), _f32),
    )(accp, y, dinv, b, g, be, Wn)


def _tc_final(accp, y, dinv, b, g, be, batch2d, lw1, lb1, lw2, lb2):
    def body(accp_ref, y_ref, dinv_ref, b_ref, g_ref, be_ref, bat_ref,
             lw1_ref, lb1_ref, lw2_ref, lb2_ref, out_ref, pooled_ref):
        h = _bn_relu(accp_ref, y_ref, dinv_ref, b_ref, g_ref, be_ref)
        bat = bat_ref[...]
        neg = _f32(-jnp.inf)

        def pool_one(gi, carry):
            row = jnp.max(jnp.where(bat == gi, h, neg), axis=0, keepdims=True)
            pooled_ref[pl.ds(gi, 1), :] = row
            return carry

        lax.fori_loop(0, G, pool_one, 0)
        pooled = pooled_ref[...]
        h2 = jnp.maximum(
            jnp.dot(pooled, lw1_ref[...], preferred_element_type=_f32) + lb1_ref[...],
            0.0)
        out_ref[...] = jnp.dot(h2, lw2_ref[...], preferred_element_type=_f32) + lb2_ref[...]

    return pl.pallas_call(
        body,
        out_shape=jax.ShapeDtypeStruct((G, 1), _f32),
        scratch_shapes=[pltpu.VMEM((G, D), _f32)],
    )(accp, y, dinv, b, g, be, batch2d, lw1, lb1, lw2, lb2)


def kernel(x, edge_attr, edge_index, batch,
           W1, b1, g1, be1, W2, b2, g2, be2, W3, b3, g3, be3,
           lw1, lb1, lw2, lb2):
    pad = EPAD - E
    srcp = jnp.concatenate(
        [edge_index[0], jnp.zeros((pad,), jnp.int32)]).reshape(NW, K, CH)
    dstp = jnp.concatenate(
        [edge_index[1], jnp.full((pad,), N, jnp.int32)]).reshape(NW, K, CH)
    srcp = jnp.concatenate(
        [srcp, jnp.zeros((NW, KAL - K, CH), jnp.int32)], axis=1)
    dstp = jnp.concatenate(
        [dstp, jnp.full((NW, KAL - K, CH), N, jnp.int32)], axis=1)
    zrows = jnp.zeros((NP, D), _f32)
    batch2d = batch.reshape(N, 1)
    b1r, g1r, be1r = b1.reshape(1, D), g1.reshape(1, D), be1.reshape(1, D)
    b2r, g2r, be2r = b2.reshape(1, D), g2.reshape(1, D), be2.reshape(1, D)
    b3r, g3r, be3r = b3.reshape(1, D), g3.reshape(1, D), be3.reshape(1, D)
    lb1r = lb1.reshape(1, D // 2)
    lb2r = lb2.reshape(1, 1)

    degp = _sc_degree(dstp, jnp.ones((CH, D), _f32), zrows)
    dinv, y1 = _tc_stage1(degp, x, W1)
    acc1 = _sc_scatter(y1, srcp, dstp, zrows)
    y2 = _tc_layer(acc1, y1, dinv, b1r, g1r, be1r, W2)
    acc2 = _sc_scatter(y2, srcp, dstp, zrows)
    y3 = _tc_layer(acc2, y2, dinv, b2r, g2r, be2r, W3)
    acc3 = _sc_scatter(y3, srcp, dstp, zrows)
    return _tc_final(acc3, y3, dinv, b3r, g3r, be3r, batch2d,
                     lw1, lb1r, lw2, lb2r)


# speedup vs baseline: 1.5638x; 1.2079x over previous
import functools

import jax
import jax.numpy as jnp
from jax import lax
from jax.experimental import pallas as pl
from jax.experimental.pallas import tpu as pltpu
from jax.experimental.pallas import tpu_sc as plsc

N = 10000
D = 128
G = 64
E = 320000
NC, NS = 2, 16
NW = NC * NS
CH = 128
K = 80
KAL = K
GRP = 2
EPAD = NW * K * CH
NP = N + 112
RPS = NP // NS
DEGW = 16

_f32 = jnp.float32


def _sc_mesh():
    return plsc.VectorSubcoreMesh(core_axis_name="c", subcore_axis_name="s")


def _sc_degree(dst_idx, ones_rows, zrows):
    @functools.partial(
        pl.kernel,
        out_type=jax.ShapeDtypeStruct((NC, NP, D), _f32),
        mesh=_sc_mesh(),
        scratch_types=[
            pltpu.VMEM((KAL, CH), jnp.int32),
            pltpu.VMEM((CH, D), _f32),
            pltpu.VMEM_SHARED((NP, D), _f32),
            pltpu.SemaphoreType.DMA,
        ],
    )
    def k(dst_hbm, ones_hbm, z_hbm, out_hbm, dst_v, ones_v, accd, sem):
        c = lax.axis_index("c")
        s = lax.axis_index("s")
        w = c * NS + s
        pltpu.sync_copy(z_hbm.at[pl.ds(s * RPS, RPS)], accd.at[pl.ds(s * RPS, RPS)])
        pltpu.sync_copy(dst_hbm.at[w], dst_v)
        pltpu.sync_copy(ones_hbm, ones_v)
        plsc.subcore_barrier()

        def body(t, carry):
            j = t * 8
            for b in range(8):
                pltpu.async_copy(ones_v, accd.at[dst_v.at[j + b]], sem, add=True)
            for b in range(8):
                pltpu.make_async_copy(ones_v, accd.at[dst_v.at[j + b]], sem).wait()
            return carry

        lax.fori_loop(0, K // 8, body, 0)
        plsc.subcore_barrier()
        pltpu.sync_copy(accd.at[pl.ds(s * RPS, RPS)],
                        out_hbm.at[c, pl.ds(s * RPS, RPS)])

    return k(dst_idx, ones_rows, zrows)


def _sc_scatter(y, src_idx, dst_idx, zrows):
    @functools.partial(
        pl.kernel,
        out_type=jax.ShapeDtypeStruct((NC, NP, D), _f32),
        mesh=_sc_mesh(),
        scratch_types=[
            pltpu.VMEM((KAL, CH), jnp.int32),
            pltpu.VMEM((KAL, CH), jnp.int32),
            pltpu.VMEM((CH, D), _f32),
            pltpu.VMEM_SHARED((NP, D), _f32),
            pltpu.SemaphoreType.DMA,
        ],
    )
    def k(y_hbm, src_hbm, dst_hbm, z_hbm, out_hbm,
          src_v, dst_v, rows_v, acc, sem):
        c = lax.axis_index("c")
        s = lax.axis_index("s")
        w = c * NS + s
        pltpu.sync_copy(z_hbm.at[pl.ds(s * RPS, RPS)], acc.at[pl.ds(s * RPS, RPS)])
        pltpu.sync_copy(src_hbm.at[w], src_v)
        pltpu.sync_copy(dst_hbm.at[w], dst_v)
        plsc.subcore_barrier()

        def body(j, carry):
            pltpu.async_copy(y_hbm.at[src_v.at[j]], rows_v, sem).wait()
            pltpu.sync_copy(rows_v, acc.at[dst_v.at[j]], add=True)
            return carry

        lax.fori_loop(0, K, body, 0)
        plsc.subcore_barrier()
        pltpu.sync_copy(acc.at[pl.ds(s * RPS, RPS)],
                        out_hbm.at[c, pl.ds(s * RPS, RPS)])

    return k(y, src_idx, dst_idx, zrows)


def _tc_stage1(degp, x, W1):
    def body(degp_ref, x_ref, w_ref, dinv_ref, y_ref):
        dp = degp_ref[...]
        deg = dp[0, :N, :1] + dp[1, :N, :1] + 1.0
        dinv = lax.rsqrt(deg)
        dinv_ref[...] = dinv
        xw = jnp.dot(x_ref[...], w_ref[...], preferred_element_type=_f32)
        y_ref[...] = xw * dinv

    return pl.pallas_call(
        body,
        out_shape=(jax.ShapeDtypeStruct((N, 1), _f32),
                   jax.ShapeDtypeStruct((N, D), _f32)),
    )(degp, x, W1)


def _bn_relu(accp_ref, y_ref, dinv_ref, b_ref, g_ref, be_ref):
    ap = accp_ref[...]
    acc = ap[0, :N, :] + ap[1, :N, :] + y_ref[...]
    r = jnp.maximum(acc * dinv_ref[...] + b_ref[...], 0.0)
    m = jnp.mean(r, axis=0, keepdims=True)
    v = jnp.mean(r * r, axis=0, keepdims=True) - m * m
    return (r - m) * lax.rsqrt(v + 1e-5) * g_ref[...] + be_ref[...]


def _tc_layer(accp, y, dinv, b, g, be, Wn):
    def body(accp_ref, y_ref, dinv_ref, b_ref, g_ref, be_ref, w_ref, yn_ref):
        h = _bn_relu(accp_ref, y_ref, dinv_ref, b_ref, g_ref, be_ref)
        yn_ref[...] = jnp.dot(h, w_ref[...], preferred_element_type=_f32) * dinv_ref[...]

    return pl.pallas_call(
        body,
        out_shape=jax.ShapeDtypeStruct((N, D), _f32),
    )(accp, y, dinv, b, g, be, Wn)


def _tc_final(accp, y, dinv, b, g, be, batch2d, lw1, lb1, lw2, lb2):
    def body(accp_ref, y_ref, dinv_ref, b_ref, g_ref, be_ref, bat_ref,
             lw1_ref, lb1_ref, lw2_ref, lb2_ref, out_ref, pooled_ref):
        h = _bn_relu(accp_ref, y_ref, dinv_ref, b_ref, g_ref, be_ref)
        bat = bat_ref[...]
        neg = _f32(-jnp.inf)

        def pool_one(gi, carry):
            row = jnp.max(jnp.where(bat == gi, h, neg), axis=0, keepdims=True)
            pooled_ref[pl.ds(gi, 1), :] = row
            return carry

        lax.fori_loop(0, G, pool_one, 0)
        pooled = pooled_ref[...]
        h2 = jnp.maximum(
            jnp.dot(pooled, lw1_ref[...], preferred_element_type=_f32) + lb1_ref[...],
            0.0)
        out_ref[...] = jnp.dot(h2, lw2_ref[...], preferred_element_type=_f32) + lb2_ref[...]

    return pl.pallas_call(
        body,
        out_shape=jax.ShapeDtypeStruct((G, 1), _f32),
        scratch_shapes=[pltpu.VMEM((G, D), _f32)],
    )(accp, y, dinv, b, g, be, batch2d, lw1, lb1, lw2, lb2)


def kernel(x, edge_attr, edge_index, batch,
           W1, b1, g1, be1, W2, b2, g2, be2, W3, b3, g3, be3,
           lw1, lb1, lw2, lb2):
    EPW = E // NW
    PW = K * CH - EPW
    pad_src = jnp.zeros((NW, PW), jnp.int32)
    pad_dst = jnp.broadcast_to(
        N + (jnp.arange(PW, dtype=jnp.int32) % (NP - N)), (NW, PW))
    srcp = jnp.concatenate(
        [edge_index[0].reshape(NW, EPW), pad_src], axis=1).reshape(NW, K, CH)
    dstp = jnp.concatenate(
        [edge_index[1].reshape(NW, EPW), pad_dst], axis=1).reshape(NW, K, CH)
    zrows = jnp.zeros((NP, D), _f32)
    batch2d = batch.reshape(N, 1)
    b1r, g1r, be1r = b1.reshape(1, D), g1.reshape(1, D), be1.reshape(1, D)
    b2r, g2r, be2r = b2.reshape(1, D), g2.reshape(1, D), be2.reshape(1, D)
    b3r, g3r, be3r = b3.reshape(1, D), g3.reshape(1, D), be3.reshape(1, D)
    lb1r = lb1.reshape(1, D // 2)
    lb2r = lb2.reshape(1, 1)

    degp = _sc_degree(dstp, jnp.ones((CH, D), _f32), zrows)
    dinv, y1 = _tc_stage1(degp, x, W1)
    acc1 = _sc_scatter(y1, srcp, dstp, zrows)
    y2 = _tc_layer(acc1, y1, dinv, b1r, g1r, be1r, W2)
    acc2 = _sc_scatter(y2, srcp, dstp, zrows)
    y3 = _tc_layer(acc2, y2, dinv, b2r, g2r, be2r, W3)
    acc3 = _sc_scatter(y3, srcp, dstp, zrows)
    return _tc_final(acc3, y3, dinv, b3r, g3r, be3r, batch2d,
                     lw1, lb1r, lw2, lb2r)


# speedup vs baseline: 2.2190x; 1.4190x over previous
import functools

import jax
import jax.numpy as jnp
from jax import lax
from jax.experimental import pallas as pl
from jax.experimental.pallas import tpu as pltpu
from jax.experimental.pallas import tpu_sc as plsc

N = 10000
D = 128
G = 64
E = 320000
NC, NS = 2, 16
NW = NC * NS
CH = 128
K = 79
KAL = K
EPAD = NW * K * CH
NP = N + 112
RPS = NP // NS
DEGW = 16

_f32 = jnp.float32


def _sc_mesh():
    return plsc.VectorSubcoreMesh(core_axis_name="c", subcore_axis_name="s")


def _sc_degree(dst_idx, ones_rows, zrows):
    @functools.partial(
        pl.kernel,
        out_type=jax.ShapeDtypeStruct((NC, NP, D), _f32),
        mesh=_sc_mesh(),
        scratch_types=[
            pltpu.VMEM((KAL, CH), jnp.int32),
            pltpu.VMEM((CH, D), _f32),
            pltpu.VMEM_SHARED((NP, D), _f32),
        ],
    )
    def k(dst_hbm, ones_hbm, z_hbm, out_hbm, dst_v, ones_v, accd):
        c = lax.axis_index("c")
        s = lax.axis_index("s")
        w = c * NS + s
        pltpu.sync_copy(z_hbm.at[pl.ds(s * RPS, RPS)], accd.at[pl.ds(s * RPS, RPS)])
        pltpu.sync_copy(dst_hbm.at[w], dst_v)
        pltpu.sync_copy(ones_hbm, ones_v)
        plsc.subcore_barrier()

        def body(j, carry):
            pltpu.sync_copy(ones_v, accd.at[dst_v.at[j]], add=True)
            return carry

        lax.fori_loop(0, K, body, 0)
        plsc.subcore_barrier()
        pltpu.sync_copy(accd.at[pl.ds(s * RPS, RPS)],
                        out_hbm.at[c, pl.ds(s * RPS, RPS)])

    return k(dst_idx, ones_rows, zrows)


def _sc_scatter(y, src_idx, dst_idx, zrows):
    @functools.partial(
        pl.kernel,
        out_type=jax.ShapeDtypeStruct((NC, NP, D), _f32),
        mesh=_sc_mesh(),
        scratch_types=[
            pltpu.VMEM((KAL, CH), jnp.int32),
            pltpu.VMEM((KAL, CH), jnp.int32),
            pltpu.VMEM((CH, D), _f32),
            pltpu.VMEM_SHARED((NP, D), _f32),
            pltpu.SemaphoreType.DMA,
        ],
    )
    def k(y_hbm, src_hbm, dst_hbm, z_hbm, out_hbm,
          src_v, dst_v, rows_v, acc, sem):
        c = lax.axis_index("c")
        s = lax.axis_index("s")
        w = c * NS + s
        pltpu.sync_copy(z_hbm.at[pl.ds(s * RPS, RPS)], acc.at[pl.ds(s * RPS, RPS)])
        pltpu.sync_copy(src_hbm.at[w], src_v)
        pltpu.sync_copy(dst_hbm.at[w], dst_v)
        plsc.subcore_barrier()

        def body(j, carry):
            pltpu.async_copy(y_hbm.at[src_v.at[j]], rows_v, sem).wait()
            pltpu.sync_copy(rows_v, acc.at[dst_v.at[j]], add=True)
            return carry

        lax.fori_loop(0, K, body, 0)
        plsc.subcore_barrier()
        pltpu.sync_copy(acc.at[pl.ds(s * RPS, RPS)],
                        out_hbm.at[c, pl.ds(s * RPS, RPS)])

    return k(y, src_idx, dst_idx, zrows)


def _tc_stage1(degp, x, W1):
    def body(degp_ref, x_ref, w_ref, dinv_ref, y_ref):
        dp = degp_ref[...]
        deg = dp[0, :N, :1] + dp[1, :N, :1] + 1.0
        dinv = lax.rsqrt(deg)
        dinv_ref[...] = dinv
        xw = jnp.dot(x_ref[...], w_ref[...], preferred_element_type=_f32)
        y_ref[...] = xw * dinv

    return pl.pallas_call(
        body,
        out_shape=(jax.ShapeDtypeStruct((N, 1), _f32),
                   jax.ShapeDtypeStruct((N, D), _f32)),
    )(degp, x, W1)


def _bn_relu(accp_ref, y_ref, dinv_ref, b_ref, g_ref, be_ref):
    ap = accp_ref[...]
    acc = ap[0, :N, :] + ap[1, :N, :] + y_ref[...]
    r = jnp.maximum(acc * dinv_ref[...] + b_ref[...], 0.0)
    m = jnp.mean(r, axis=0, keepdims=True)
    v = jnp.mean(r * r, axis=0, keepdims=True) - m * m
    return (r - m) * lax.rsqrt(v + 1e-5) * g_ref[...] + be_ref[...]


def _tc_layer(accp, y, dinv, b, g, be, Wn):
    def body(accp_ref, y_ref, dinv_ref, b_ref, g_ref, be_ref, w_ref, yn_ref):
        h = _bn_relu(accp_ref, y_ref, dinv_ref, b_ref, g_ref, be_ref)
        yn_ref[...] = jnp.dot(h, w_ref[...], preferred_element_type=_f32) * dinv_ref[...]

    return pl.pallas_call(
        body,
        out_shape=jax.ShapeDtypeStruct((N, D), _f32),
    )(accp, y, dinv, b, g, be, Wn)


def _tc_final(accp, y, dinv, b, g, be, batch2d, lw1, lb1, lw2, lb2):
    def body(accp_ref, y_ref, dinv_ref, b_ref, g_ref, be_ref, bat_ref,
             lw1_ref, lb1_ref, lw2_ref, lb2_ref, out_ref, pooled_ref):
        h = _bn_relu(accp_ref, y_ref, dinv_ref, b_ref, g_ref, be_ref)
        bat = bat_ref[...]
        neg = _f32(-jnp.inf)

        def pool_one(gi, carry):
            row = jnp.max(jnp.where(bat == gi, h, neg), axis=0, keepdims=True)
            pooled_ref[pl.ds(gi, 1), :] = row
            return carry

        lax.fori_loop(0, G, pool_one, 0)
        pooled = pooled_ref[...]
        h2 = jnp.maximum(
            jnp.dot(pooled, lw1_ref[...], preferred_element_type=_f32) + lb1_ref[...],
            0.0)
        out_ref[...] = jnp.dot(h2, lw2_ref[...], preferred_element_type=_f32) + lb2_ref[...]

    return pl.pallas_call(
        body,
        out_shape=jax.ShapeDtypeStruct((G, 1), _f32),
        scratch_shapes=[pltpu.VMEM((G, D), _f32)],
    )(accp, y, dinv, b, g, be, batch2d, lw1, lb1, lw2, lb2)


def kernel(x, edge_attr, edge_index, batch,
           W1, b1, g1, be1, W2, b2, g2, be2, W3, b3, g3, be3,
           lw1, lb1, lw2, lb2):
    pad = EPAD - E
    srcp = jnp.concatenate(
        [edge_index[0], jnp.zeros((pad,), jnp.int32)]).reshape(NW, K, CH)
    dstp = jnp.concatenate(
        [edge_index[1], jnp.full((pad,), N, jnp.int32)]).reshape(NW, K, CH)
    zrows = jnp.zeros((NP, D), _f32)
    batch2d = batch.reshape(N, 1)
    b1r, g1r, be1r = b1.reshape(1, D), g1.reshape(1, D), be1.reshape(1, D)
    b2r, g2r, be2r = b2.reshape(1, D), g2.reshape(1, D), be2.reshape(1, D)
    b3r, g3r, be3r = b3.reshape(1, D), g3.reshape(1, D), be3.reshape(1, D)
    lb1r = lb1.reshape(1, D // 2)
    lb2r = lb2.reshape(1, 1)

    degp = _sc_degree(dstp, jnp.ones((CH, D), _f32), zrows)
    dinv, y1 = _tc_stage1(degp, x, W1)
    acc1 = _sc_scatter(y1, srcp, dstp, zrows)
    y2 = _tc_layer(acc1, y1, dinv, b1r, g1r, be1r, W2)
    acc2 = _sc_scatter(y2, srcp, dstp, zrows)
    y3 = _tc_layer(acc2, y2, dinv, b2r, g2r, be2r, W3)
    acc3 = _sc_scatter(y3, srcp, dstp, zrows)
    return _tc_final(acc3, y3, dinv, b3r, g3r, be3r, batch2d,
                     lw1, lb1r, lw2, lb2r)


# speedup vs baseline: 2.2197x; 1.0003x over previous
import functools

import jax
import jax.numpy as jnp
from jax import lax
from jax.experimental import pallas as pl
from jax.experimental.pallas import tpu as pltpu
from jax.experimental.pallas import tpu_sc as plsc

N = 10000
D = 128
G = 64
E = 320000
NC, NS = 2, 16
NW = NC * NS
CH = 128
K = 79
KAL = K
EPAD = NW * K * CH
NP = N + 112
RPS = NP // NS
DEGW = 16

_f32 = jnp.float32


def _sc_mesh():
    return plsc.VectorSubcoreMesh(core_axis_name="c", subcore_axis_name="s")


def _sc_degree(dst_idx, ones_rows, zrows):
    @functools.partial(
        pl.kernel,
        out_type=jax.ShapeDtypeStruct((NC, NP, D), _f32),
        mesh=_sc_mesh(),
        scratch_types=[
            pltpu.VMEM((KAL, CH), jnp.int32),
            pltpu.VMEM((CH, D), _f32),
            pltpu.VMEM_SHARED((NP, D), _f32),
        ],
    )
    def k(dst_hbm, ones_hbm, z_hbm, out_hbm, dst_v, ones_v, accd):
        c = lax.axis_index("c")
        s = lax.axis_index("s")
        w = c * NS + s
        pltpu.sync_copy(z_hbm.at[pl.ds(s * RPS, RPS)], accd.at[pl.ds(s * RPS, RPS)])
        pltpu.sync_copy(dst_hbm.at[w], dst_v)
        pltpu.sync_copy(ones_hbm, ones_v)
        plsc.subcore_barrier()

        def body(j, carry):
            pltpu.sync_copy(ones_v, accd.at[dst_v.at[j]], add=True)
            return carry

        lax.fori_loop(0, K, body, 0)
        plsc.subcore_barrier()
        pltpu.sync_copy(accd.at[pl.ds(s * RPS, RPS)],
                        out_hbm.at[c, pl.ds(s * RPS, RPS)])

    return k(dst_idx, ones_rows, zrows)


def _sc_scatter(y, src_idx, dst_idx, zrows):
    @functools.partial(
        pl.kernel,
        out_type=jax.ShapeDtypeStruct((NC, NP, D), _f32),
        mesh=_sc_mesh(),
        scratch_types=[
            pltpu.VMEM((KAL, CH), jnp.int32),
            pltpu.VMEM((KAL, CH), jnp.int32),
            pltpu.VMEM((CH, D), _f32),
            pltpu.VMEM_SHARED((NP, D), _f32),
            pltpu.SemaphoreType.DMA,
        ],
    )
    def k(y_hbm, src_hbm, dst_hbm, z_hbm, out_hbm,
          src_v, dst_v, rows_v, acc, sem):
        c = lax.axis_index("c")
        s = lax.axis_index("s")
        w = c * NS + s
        pltpu.sync_copy(z_hbm.at[pl.ds(s * RPS, RPS)], acc.at[pl.ds(s * RPS, RPS)])
        pltpu.sync_copy(src_hbm.at[w], src_v)
        pltpu.sync_copy(dst_hbm.at[w], dst_v)
        plsc.subcore_barrier()

        def body(j, carry):
            pltpu.async_copy(y_hbm.at[src_v.at[j]], rows_v, sem).wait()
            pltpu.sync_copy(rows_v, acc.at[dst_v.at[j]], add=True)
            return carry

        lax.fori_loop(0, K, body, 0)
        plsc.subcore_barrier()
        pltpu.sync_copy(acc.at[pl.ds(s * RPS, RPS)],
                        out_hbm.at[c, pl.ds(s * RPS, RPS)])

    return k(y, src_idx, dst_idx, zrows)


def _tc_stage1(degp, x, W1):
    def body(degp_ref, x_ref, w_ref, dinv_ref, y_ref):
        dp = degp_ref[...]
        deg = dp[0, :N, :1] + dp[1, :N, :1] + 1.0
        dinv = lax.rsqrt(deg)
        dinv_ref[...] = dinv
        xw = jnp.dot(x_ref[...], w_ref[...], preferred_element_type=_f32)
        y_ref[...] = xw * dinv

    return pl.pallas_call(
        body,
        out_shape=(jax.ShapeDtypeStruct((N, 1), _f32),
                   jax.ShapeDtypeStruct((N, D), _f32)),
    )(degp, x, W1)


def _bn_relu(accp_ref, y_ref, dinv_ref, b_ref, g_ref, be_ref):
    ap = accp_ref[...]
    acc = ap[0, :N, :] + ap[1, :N, :] + y_ref[...]
    r = jnp.maximum(acc * dinv_ref[...] + b_ref[...], 0.0)
    m = jnp.mean(r, axis=0, keepdims=True)
    v = jnp.mean(r * r, axis=0, keepdims=True) - m * m
    return (r - m) * lax.rsqrt(v + 1e-5) * g_ref[...] + be_ref[...]


def _tc_layer(accp, y, dinv, b, g, be, Wn):
    def body(accp_ref, y_ref, dinv_ref, b_ref, g_ref, be_ref, w_ref, yn_ref):
        h = _bn_relu(accp_ref, y_ref, dinv_ref, b_ref, g_ref, be_ref)
        yn_ref[...] = jnp.dot(h, w_ref[...], preferred_element_type=_f32) * dinv_ref[...]

    return pl.pallas_call(
        body,
        out_shape=jax.ShapeDtypeStruct((N, D), _f32),
    )(accp, y, dinv, b, g, be, Wn)


def _tc_final(accp, y, dinv, b, g, be, batch2d, lw1, lb1, lw2, lb2):
    def body(accp_ref, y_ref, dinv_ref, b_ref, g_ref, be_ref, bat_ref,
             lw1_ref, lb1_ref, lw2_ref, lb2_ref, out_ref, pooled_ref):
        h = _bn_relu(accp_ref, y_ref, dinv_ref, b_ref, g_ref, be_ref)
        bat = bat_ref[...]
        neg = _f32(-jnp.inf)

        def pool_one(gi, carry):
            row = jnp.max(jnp.where(bat == gi, h, neg), axis=0, keepdims=True)
            pooled_ref[pl.ds(gi, 1), :] = row
            return carry

        lax.fori_loop(0, G, pool_one, 0)
        pooled = pooled_ref[...]
        h2 = jnp.maximum(
            jnp.dot(pooled, lw1_ref[...], preferred_element_type=_f32) + lb1_ref[...],
            0.0)
        out_ref[...] = jnp.dot(h2, lw2_ref[...], preferred_element_type=_f32) + lb2_ref[...]

    return pl.pallas_call(
        body,
        out_shape=jax.ShapeDtypeStruct((G, 1), _f32),
        scratch_shapes=[pltpu.VMEM((G, D), _f32)],
    )(accp, y, dinv, b, g, be, batch2d, lw1, lb1, lw2, lb2)


def kernel(x, edge_attr, edge_index, batch,
           W1, b1, g1, be1, W2, b2, g2, be2, W3, b3, g3, be3,
           lw1, lb1, lw2, lb2):
    pad = EPAD - E
    pad_dst = N + (jnp.arange(pad, dtype=jnp.int32) % (NP - N))
    srcp = jnp.concatenate(
        [edge_index[0], jnp.zeros((pad,), jnp.int32)]).reshape(NW, K, CH)
    dstp = jnp.concatenate(
        [edge_index[1], pad_dst]).reshape(NW, K, CH)
    zrows = jnp.zeros((NP, D), _f32)
    batch2d = batch.reshape(N, 1)
    b1r, g1r, be1r = b1.reshape(1, D), g1.reshape(1, D), be1.reshape(1, D)
    b2r, g2r, be2r = b2.reshape(1, D), g2.reshape(1, D), be2.reshape(1, D)
    b3r, g3r, be3r = b3.reshape(1, D), g3.reshape(1, D), be3.reshape(1, D)
    lb1r = lb1.reshape(1, D // 2)
    lb2r = lb2.reshape(1, 1)

    degp = _sc_degree(dstp, jnp.ones((CH, D), _f32), zrows)
    dinv, y1 = _tc_stage1(degp, x, W1)
    acc1 = _sc_scatter(y1, srcp, dstp, zrows)
    y2 = _tc_layer(acc1, y1, dinv, b1r, g1r, be1r, W2)
    acc2 = _sc_scatter(y2, srcp, dstp, zrows)
    y3 = _tc_layer(acc2, y2, dinv, b2r, g2r, be2r, W3)
    acc3 = _sc_scatter(y3, srcp, dstp, zrows)
    return _tc_final(acc3, y3, dinv, b3r, g3r, be3r, batch2d,
                     lw1, lb1r, lw2, lb2r)


# speedup vs baseline: 2.3347x; 1.0518x over previous
import functools

import jax
import jax.numpy as jnp
from jax import lax
from jax.experimental import pallas as pl
from jax.experimental.pallas import tpu as pltpu
from jax.experimental.pallas import tpu_sc as plsc

N = 10000
D = 128
G = 64
E = 320000
NC, NS = 2, 16
NW = NC * NS
CH = 128
K = 79
KAL = K
HK = 40
EPAD = NW * K * CH
NP = N + 112
RPS = NP // NS
DEGW = 16

_f32 = jnp.float32


def _sc_mesh():
    return plsc.VectorSubcoreMesh(core_axis_name="c", subcore_axis_name="s")


def _sc_degree(dst_idx, ones_rows, zrows):
    @functools.partial(
        pl.kernel,
        out_type=jax.ShapeDtypeStruct((NC, NP, D), _f32),
        mesh=_sc_mesh(),
        scratch_types=[
            pltpu.VMEM((KAL, CH), jnp.int32),
            pltpu.VMEM((CH, D), _f32),
            pltpu.VMEM_SHARED((NP, D), _f32),
        ],
    )
    def k(dst_hbm, ones_hbm, z_hbm, out_hbm, dst_v, ones_v, accd):
        c = lax.axis_index("c")
        s = lax.axis_index("s")
        w = c * NS + s
        pltpu.sync_copy(z_hbm.at[pl.ds(s * RPS, RPS)], accd.at[pl.ds(s * RPS, RPS)])
        pltpu.sync_copy(dst_hbm.at[w], dst_v)
        pltpu.sync_copy(ones_hbm, ones_v)
        plsc.subcore_barrier()

        def body(j, carry):
            pltpu.sync_copy(ones_v, accd.at[dst_v.at[j]], add=True)
            return carry

        lax.fori_loop(0, K, body, 0)
        plsc.subcore_barrier()
        pltpu.sync_copy(accd.at[pl.ds(s * RPS, RPS)],
                        out_hbm.at[c, pl.ds(s * RPS, RPS)])

    return k(dst_idx, ones_rows, zrows)


def _sc_scatter(y, src_idx, dst_idx, zrows):
    @functools.partial(
        pl.kernel,
        out_type=jax.ShapeDtypeStruct((NC, NP, D), _f32),
        mesh=_sc_mesh(),
        scratch_types=[
            pltpu.VMEM((HK, CH), jnp.int32),
            pltpu.VMEM((HK, CH), jnp.int32),
            pltpu.VMEM((CH, D), _f32),
            pltpu.VMEM((CH, D), _f32),
            pltpu.VMEM_SHARED((NP, D), _f32),
            pltpu.SemaphoreType.DMA,
            pltpu.SemaphoreType.DMA,
        ],
    )
    def k(y_hbm, src_hbm, dst_hbm, z_hbm, out_hbm,
          src_v, dst_v, rows0, rows1, acc, sem0, sem1):
        c = lax.axis_index("c")
        s = lax.axis_index("s")
        w = c * NS + s
        pltpu.sync_copy(z_hbm.at[pl.ds(s * RPS, RPS)], acc.at[pl.ds(s * RPS, RPS)])
        plsc.subcore_barrier()

        def pair_body(t, carry):
            j = t * 2
            d0 = pltpu.async_copy(y_hbm.at[src_v.at[j]], rows0, sem0)
            d1 = pltpu.async_copy(y_hbm.at[src_v.at[j + 1]], rows1, sem1)
            d0.wait()
            pltpu.sync_copy(rows0, acc.at[dst_v.at[j]], add=True)
            d1.wait()
            pltpu.sync_copy(rows1, acc.at[dst_v.at[j + 1]], add=True)
            return carry

        pltpu.sync_copy(src_hbm.at[w, pl.ds(0, HK)], src_v)
        pltpu.sync_copy(dst_hbm.at[w, pl.ds(0, HK)], dst_v)
        lax.fori_loop(0, HK // 2, pair_body, 0)
        pltpu.sync_copy(src_hbm.at[w, pl.ds(HK, K - HK)],
                        src_v.at[pl.ds(0, K - HK)])
        pltpu.sync_copy(dst_hbm.at[w, pl.ds(HK, K - HK)],
                        dst_v.at[pl.ds(0, K - HK)])
        lax.fori_loop(0, (K - HK) // 2, pair_body, 0)
        pltpu.async_copy(y_hbm.at[src_v.at[K - HK - 1]], rows0, sem0).wait()
        pltpu.sync_copy(rows0, acc.at[dst_v.at[K - HK - 1]], add=True)
        plsc.subcore_barrier()
        pltpu.sync_copy(acc.at[pl.ds(s * RPS, RPS)],
                        out_hbm.at[c, pl.ds(s * RPS, RPS)])

    return k(y, src_idx, dst_idx, zrows)


def _tc_stage1(degp, x, W1):
    def body(degp_ref, x_ref, w_ref, dinv_ref, y_ref):
        dp = degp_ref[...]
        deg = dp[0, :N, :1] + dp[1, :N, :1] + 1.0
        dinv = lax.rsqrt(deg)
        dinv_ref[...] = dinv
        xw = jnp.dot(x_ref[...], w_ref[...], preferred_element_type=_f32)
        y_ref[...] = xw * dinv

    return pl.pallas_call(
        body,
        out_shape=(jax.ShapeDtypeStruct((N, 1), _f32),
                   jax.ShapeDtypeStruct((N, D), _f32)),
    )(degp, x, W1)


def _bn_relu(accp_ref, y_ref, dinv_ref, b_ref, g_ref, be_ref):
    ap = accp_ref[...]
    acc = ap[0, :N, :] + ap[1, :N, :] + y_ref[...]
    r = jnp.maximum(acc * dinv_ref[...] + b_ref[...], 0.0)
    m = jnp.mean(r, axis=0, keepdims=True)
    v = jnp.mean(r * r, axis=0, keepdims=True) - m * m
    return (r - m) * lax.rsqrt(v + 1e-5) * g_ref[...] + be_ref[...]


def _tc_layer(accp, y, dinv, b, g, be, Wn):
    def body(accp_ref, y_ref, dinv_ref, b_ref, g_ref, be_ref, w_ref, yn_ref):
        h = _bn_relu(accp_ref, y_ref, dinv_ref, b_ref, g_ref, be_ref)
        yn_ref[...] = jnp.dot(h, w_ref[...], preferred_element_type=_f32) * dinv_ref[...]

    return pl.pallas_call(
        body,
        out_shape=jax.ShapeDtypeStruct((N, D), _f32),
    )(accp, y, dinv, b, g, be, Wn)


def _tc_final(accp, y, dinv, b, g, be, batch2d, lw1, lb1, lw2, lb2):
    def body(accp_ref, y_ref, dinv_ref, b_ref, g_ref, be_ref, bat_ref,
             lw1_ref, lb1_ref, lw2_ref, lb2_ref, out_ref, pooled_ref):
        h = _bn_relu(accp_ref, y_ref, dinv_ref, b_ref, g_ref, be_ref)
        bat = bat_ref[...]
        neg = _f32(-jnp.inf)

        def pool_one(gi, carry):
            row = jnp.max(jnp.where(bat == gi, h, neg), axis=0, keepdims=True)
            pooled_ref[pl.ds(gi, 1), :] = row
            return carry

        lax.fori_loop(0, G, pool_one, 0)
        pooled = pooled_ref[...]
        h2 = jnp.maximum(
            jnp.dot(pooled, lw1_ref[...], preferred_element_type=_f32) + lb1_ref[...],
            0.0)
        out_ref[...] = jnp.dot(h2, lw2_ref[...], preferred_element_type=_f32) + lb2_ref[...]

    return pl.pallas_call(
        body,
        out_shape=jax.ShapeDtypeStruct((G, 1), _f32),
        scratch_shapes=[pltpu.VMEM((G, D), _f32)],
    )(accp, y, dinv, b, g, be, batch2d, lw1, lb1, lw2, lb2)


def kernel(x, edge_attr, edge_index, batch,
           W1, b1, g1, be1, W2, b2, g2, be2, W3, b3, g3, be3,
           lw1, lb1, lw2, lb2):
    pad = EPAD - E
    pad_dst = N + (jnp.arange(pad, dtype=jnp.int32) % (NP - N))
    srcp = jnp.concatenate(
        [edge_index[0], jnp.zeros((pad,), jnp.int32)]).reshape(NW, K, CH)
    dstp = jnp.concatenate(
        [edge_index[1], pad_dst]).reshape(NW, K, CH)
    zrows = jnp.zeros((NP, D), _f32)
    batch2d = batch.reshape(N, 1)
    b1r, g1r, be1r = b1.reshape(1, D), g1.reshape(1, D), be1.reshape(1, D)
    b2r, g2r, be2r = b2.reshape(1, D), g2.reshape(1, D), be2.reshape(1, D)
    b3r, g3r, be3r = b3.reshape(1, D), g3.reshape(1, D), be3.reshape(1, D)
    lb1r = lb1.reshape(1, D // 2)
    lb2r = lb2.reshape(1, 1)

    degp = _sc_degree(dstp, jnp.ones((CH, D), _f32), zrows)
    dinv, y1 = _tc_stage1(degp, x, W1)
    acc1 = _sc_scatter(y1, srcp, dstp, zrows)
    y2 = _tc_layer(acc1, y1, dinv, b1r, g1r, be1r, W2)
    acc2 = _sc_scatter(y2, srcp, dstp, zrows)
    y3 = _tc_layer(acc2, y2, dinv, b2r, g2r, be2r, W3)
    acc3 = _sc_scatter(y3, srcp, dstp, zrows)
    return _tc_final(acc3, y3, dinv, b3r, g3r, be3r, batch2d,
                     lw1, lb1r, lw2, lb2r)


# speedup vs baseline: 2.3527x; 1.0077x over previous
import functools

import jax
import jax.numpy as jnp
from jax import lax
from jax.experimental import pallas as pl
from jax.experimental.pallas import tpu as pltpu
from jax.experimental.pallas import tpu_sc as plsc

N = 10000
D = 128
G = 64
E = 320000
NC, NS = 2, 16
NW = NC * NS
CH = 128
K = 79
KAL = K
HK = 40
EPAD = NW * K * CH
NP = N + 112
RPS = NP // NS
DEGW = 16

_f32 = jnp.float32


def _sc_mesh():
    return plsc.VectorSubcoreMesh(core_axis_name="c", subcore_axis_name="s")


def _sc_degree(dst_idx, ones_rows, zrows):
    @functools.partial(
        pl.kernel,
        out_type=jax.ShapeDtypeStruct((NC, NP, D), _f32),
        mesh=_sc_mesh(),
        scratch_types=[
            pltpu.VMEM((KAL, CH), jnp.int32),
            pltpu.VMEM((CH, D), _f32),
            pltpu.VMEM_SHARED((NP, D), _f32),
        ],
    )
    def k(dst_hbm, ones_hbm, z_hbm, out_hbm, dst_v, ones_v, accd):
        c = lax.axis_index("c")
        s = lax.axis_index("s")
        w = c * NS + s
        pltpu.sync_copy(z_hbm.at[pl.ds(s * RPS, RPS)], accd.at[pl.ds(s * RPS, RPS)])
        pltpu.sync_copy(dst_hbm.at[w], dst_v)
        pltpu.sync_copy(ones_hbm, ones_v)
        plsc.subcore_barrier()

        def body(j, carry):
            pltpu.sync_copy(ones_v, accd.at[dst_v.at[j]], add=True)
            return carry

        lax.fori_loop(0, K, body, 0)
        plsc.subcore_barrier()
        pltpu.sync_copy(accd.at[pl.ds(s * RPS, RPS)],
                        out_hbm.at[c, pl.ds(s * RPS, RPS)])

    return k(dst_idx, ones_rows, zrows)


def _sc_scatter(y, src_idx, dst_idx, zrows):
    @functools.partial(
        pl.kernel,
        out_type=jax.ShapeDtypeStruct((NC, NP, D), _f32),
        mesh=_sc_mesh(),
        scratch_types=[
            pltpu.VMEM((HK, CH), jnp.int32),
            pltpu.VMEM((HK, CH), jnp.int32),
            pltpu.VMEM((CH, D), _f32),
            pltpu.VMEM((CH, D), _f32),
            pltpu.VMEM_SHARED((NP, D), _f32),
            pltpu.SemaphoreType.DMA,
            pltpu.SemaphoreType.DMA,
        ],
    )
    def k(y_hbm, src_hbm, dst_hbm, z_hbm, out_hbm,
          src_v, dst_v, rows0, rows1, acc, sem0, sem1):
        c = lax.axis_index("c")
        s = lax.axis_index("s")
        w = c * NS + s
        pltpu.sync_copy(z_hbm.at[pl.ds(s * RPS, RPS)], acc.at[pl.ds(s * RPS, RPS)])
        plsc.subcore_barrier()

        def pair_body(t, carry):
            j = t * 2
            d0 = pltpu.async_copy(y_hbm.at[src_v.at[j]], rows0, sem0)
            d1 = pltpu.async_copy(y_hbm.at[src_v.at[j + 1]], rows1, sem1)
            d0.wait()
            s0 = pltpu.async_copy(rows0, acc.at[dst_v.at[j]], sem0, add=True)
            d1.wait()
            s1 = pltpu.async_copy(rows1, acc.at[dst_v.at[j + 1]], sem1, add=True)
            s0.wait()
            s1.wait()
            return carry

        pltpu.sync_copy(src_hbm.at[w, pl.ds(0, HK)], src_v)
        pltpu.sync_copy(dst_hbm.at[w, pl.ds(0, HK)], dst_v)
        lax.fori_loop(0, HK // 2, pair_body, 0)
        pltpu.sync_copy(src_hbm.at[w, pl.ds(HK, K - HK)],
                        src_v.at[pl.ds(0, K - HK)])
        pltpu.sync_copy(dst_hbm.at[w, pl.ds(HK, K - HK)],
                        dst_v.at[pl.ds(0, K - HK)])
        lax.fori_loop(0, (K - HK) // 2, pair_body, 0)
        pltpu.async_copy(y_hbm.at[src_v.at[K - HK - 1]], rows0, sem0).wait()
        pltpu.sync_copy(rows0, acc.at[dst_v.at[K - HK - 1]], add=True)
        plsc.subcore_barrier()
        pltpu.sync_copy(acc.at[pl.ds(s * RPS, RPS)],
                        out_hbm.at[c, pl.ds(s * RPS, RPS)])

    return k(y, src_idx, dst_idx, zrows)


def _tc_stage1(degp, x, W1):
    def body(degp_ref, x_ref, w_ref, dinv_ref, y_ref):
        dp = degp_ref[...]
        deg = dp[0, :N, :1] + dp[1, :N, :1] + 1.0
        dinv = lax.rsqrt(deg)
        dinv_ref[...] = dinv
        xw = jnp.dot(x_ref[...], w_ref[...], preferred_element_type=_f32)
        y_ref[...] = xw * dinv

    return pl.pallas_call(
        body,
        out_shape=(jax.ShapeDtypeStruct((N, 1), _f32),
                   jax.ShapeDtypeStruct((N, D), _f32)),
    )(degp, x, W1)


def _bn_relu(accp_ref, y_ref, dinv_ref, b_ref, g_ref, be_ref):
    ap = accp_ref[...]
    acc = ap[0, :N, :] + ap[1, :N, :] + y_ref[...]
    r = jnp.maximum(acc * dinv_ref[...] + b_ref[...], 0.0)
    m = jnp.mean(r, axis=0, keepdims=True)
    v = jnp.mean(r * r, axis=0, keepdims=True) - m * m
    return (r - m) * lax.rsqrt(v + 1e-5) * g_ref[...] + be_ref[...]


def _tc_layer(accp, y, dinv, b, g, be, Wn):
    def body(accp_ref, y_ref, dinv_ref, b_ref, g_ref, be_ref, w_ref, yn_ref):
        h = _bn_relu(accp_ref, y_ref, dinv_ref, b_ref, g_ref, be_ref)
        yn_ref[...] = jnp.dot(h, w_ref[...], preferred_element_type=_f32) * dinv_ref[...]

    return pl.pallas_call(
        body,
        out_shape=jax.ShapeDtypeStruct((N, D), _f32),
    )(accp, y, dinv, b, g, be, Wn)


def _tc_final(accp, y, dinv, b, g, be, batch2d, lw1, lb1, lw2, lb2):
    def body(accp_ref, y_ref, dinv_ref, b_ref, g_ref, be_ref, bat_ref,
             lw1_ref, lb1_ref, lw2_ref, lb2_ref, out_ref, pooled_ref):
        h = _bn_relu(accp_ref, y_ref, dinv_ref, b_ref, g_ref, be_ref)
        bat = bat_ref[...]
        neg = _f32(-jnp.inf)

        def pool_one(gi, carry):
            row = jnp.max(jnp.where(bat == gi, h, neg), axis=0, keepdims=True)
            pooled_ref[pl.ds(gi, 1), :] = row
            return carry

        lax.fori_loop(0, G, pool_one, 0)
        pooled = pooled_ref[...]
        h2 = jnp.maximum(
            jnp.dot(pooled, lw1_ref[...], preferred_element_type=_f32) + lb1_ref[...],
            0.0)
        out_ref[...] = jnp.dot(h2, lw2_ref[...], preferred_element_type=_f32) + lb2_ref[...]

    return pl.pallas_call(
        body,
        out_shape=jax.ShapeDtypeStruct((G, 1), _f32),
        scratch_shapes=[pltpu.VMEM((G, D), _f32)],
    )(accp, y, dinv, b, g, be, batch2d, lw1, lb1, lw2, lb2)


def kernel(x, edge_attr, edge_index, batch,
           W1, b1, g1, be1, W2, b2, g2, be2, W3, b3, g3, be3,
           lw1, lb1, lw2, lb2):
    pad = EPAD - E
    pad_dst = N + (jnp.arange(pad, dtype=jnp.int32) % (NP - N))
    srcp = jnp.concatenate(
        [edge_index[0], jnp.zeros((pad,), jnp.int32)]).reshape(NW, K, CH)
    dstp = jnp.concatenate(
        [edge_index[1], pad_dst]).reshape(NW, K, CH)
    zrows = jnp.zeros((NP, D), _f32)
    batch2d = batch.reshape(N, 1)
    b1r, g1r, be1r = b1.reshape(1, D), g1.reshape(1, D), be1.reshape(1, D)
    b2r, g2r, be2r = b2.reshape(1, D), g2.reshape(1, D), be2.reshape(1, D)
    b3r, g3r, be3r = b3.reshape(1, D), g3.reshape(1, D), be3.reshape(1, D)
    lb1r = lb1.reshape(1, D // 2)
    lb2r = lb2.reshape(1, 1)

    degp = _sc_degree(dstp, jnp.ones((CH, D), _f32), zrows)
    dinv, y1 = _tc_stage1(degp, x, W1)
    acc1 = _sc_scatter(y1, srcp, dstp, zrows)
    y2 = _tc_layer(acc1, y1, dinv, b1r, g1r, be1r, W2)
    acc2 = _sc_scatter(y2, srcp, dstp, zrows)
    y3 = _tc_layer(acc2, y2, dinv, b2r, g2r, be2r, W3)
    acc3 = _sc_scatter(y3, srcp, dstp, zrows)
    return _tc_final(acc3, y3, dinv, b3r, g3r, be3r, batch2d,
                     lw1, lb1r, lw2, lb2r)


# speedup vs baseline: 2.3715x; 1.0080x over previous
import functools

import jax
import jax.numpy as jnp
from jax import lax
from jax.experimental import pallas as pl
from jax.experimental.pallas import tpu as pltpu
from jax.experimental.pallas import tpu_sc as plsc

N = 10000
D = 128
G = 64
E = 320000
NC, NS = 2, 16
NW = NC * NS
CH = 128
K = 79
KAL = K
HK = 40
EPAD = NW * K * CH
NP = N + 112
RPS = NP // NS
DEGW = 16

_f32 = jnp.float32


def _sc_mesh():
    return plsc.VectorSubcoreMesh(core_axis_name="c", subcore_axis_name="s")


def _sc_degree(dst_idx, ones_rows, zrows):
    @functools.partial(
        pl.kernel,
        out_type=jax.ShapeDtypeStruct((NC, NP, D), _f32),
        mesh=_sc_mesh(),
        scratch_types=[
            pltpu.VMEM((KAL, CH), jnp.int32),
            pltpu.VMEM((CH, D), _f32),
            pltpu.VMEM_SHARED((NP, D), _f32),
        ],
    )
    def k(dst_hbm, ones_hbm, z_hbm, out_hbm, dst_v, ones_v, accd):
        c = lax.axis_index("c")
        s = lax.axis_index("s")
        w = c * NS + s
        pltpu.sync_copy(z_hbm.at[pl.ds(s * RPS, RPS)], accd.at[pl.ds(s * RPS, RPS)])
        pltpu.sync_copy(dst_hbm.at[w], dst_v)
        pltpu.sync_copy(ones_hbm, ones_v)
        plsc.subcore_barrier()

        def body(j, carry):
            pltpu.sync_copy(ones_v, accd.at[dst_v.at[j]], add=True)
            return carry

        lax.fori_loop(0, K, body, 0)
        plsc.subcore_barrier()
        pltpu.sync_copy(accd.at[pl.ds(s * RPS, RPS)],
                        out_hbm.at[c, pl.ds(s * RPS, RPS)])

    return k(dst_idx, ones_rows, zrows)


def _sc_scatter(y, src_idx, dst_idx, zrows):
    @functools.partial(
        pl.kernel,
        out_type=jax.ShapeDtypeStruct((NC, NP, D), _f32),
        mesh=_sc_mesh(),
        scratch_types=[
            pltpu.VMEM((HK, CH), jnp.int32),
            pltpu.VMEM((HK, CH), jnp.int32),
            pltpu.VMEM((CH, D), _f32),
            pltpu.VMEM((CH, D), _f32),
            pltpu.VMEM_SHARED((NP, D), _f32),
            pltpu.SemaphoreType.DMA,
            pltpu.SemaphoreType.DMA,
            pltpu.SemaphoreType.DMA,
            pltpu.SemaphoreType.DMA,
        ],
    )
    def k(y_hbm, src_hbm, dst_hbm, z_hbm, out_hbm,
          src_v, dst_v, rows0, rows1, acc, g0, g1, ss0, ss1):
        c = lax.axis_index("c")
        s = lax.axis_index("s")
        w = c * NS + s
        pltpu.sync_copy(z_hbm.at[pl.ds(s * RPS, RPS)], acc.at[pl.ds(s * RPS, RPS)])
        plsc.subcore_barrier()

        def fire_g(rows, sem, j):
            pltpu.async_copy(y_hbm.at[src_v.at[j]], rows, sem)

        def wait_g(rows, sem):
            pltpu.make_async_copy(y_hbm.at[src_v.at[0]], rows, sem).wait()

        def fire_s(rows, sem, j):
            pltpu.async_copy(rows, acc.at[dst_v.at[j]], sem, add=True)

        def wait_s(rows, sem, j):
            pltpu.make_async_copy(rows, acc.at[dst_v.at[j]], sem).wait()

        def run_phase(npairs):
            last = 2 * npairs - 1
            fire_g(rows0, g0, 0)
            fire_g(rows1, g1, 1)

            def body(t, carry):
                j = 2 * t
                wait_g(rows0, g0)
                fire_s(rows0, ss0, j)
                wait_g(rows1, g1)
                fire_s(rows1, ss1, j + 1)
                wait_s(rows0, ss0, j)
                fire_g(rows0, g0, jnp.minimum(j + 2, last))
                wait_s(rows1, ss1, j + 1)
                fire_g(rows1, g1, jnp.minimum(j + 3, last))
                return carry

            lax.fori_loop(0, npairs, body, 0)
            wait_g(rows0, g0)
            wait_g(rows1, g1)

        pltpu.sync_copy(src_hbm.at[w, pl.ds(0, HK)], src_v)
        pltpu.sync_copy(dst_hbm.at[w, pl.ds(0, HK)], dst_v)
        run_phase(HK // 2)
        pltpu.sync_copy(src_hbm.at[w, pl.ds(HK, K - HK)],
                        src_v.at[pl.ds(0, K - HK)])
        pltpu.sync_copy(dst_hbm.at[w, pl.ds(HK, K - HK)],
                        dst_v.at[pl.ds(0, K - HK)])
        run_phase((K - HK) // 2)
        pltpu.async_copy(y_hbm.at[src_v.at[K - HK - 1]], rows0, g0).wait()
        pltpu.sync_copy(rows0, acc.at[dst_v.at[K - HK - 1]], add=True)
        plsc.subcore_barrier()
        pltpu.sync_copy(acc.at[pl.ds(s * RPS, RPS)],
                        out_hbm.at[c, pl.ds(s * RPS, RPS)])

    return k(y, src_idx, dst_idx, zrows)


def _tc_stage1(degp, x, W1):
    def body(degp_ref, x_ref, w_ref, dinv_ref, y_ref):
        dp = degp_ref[...]
        deg = dp[0, :N, :1] + dp[1, :N, :1] + 1.0
        dinv = lax.rsqrt(deg)
        dinv_ref[...] = dinv
        xw = jnp.dot(x_ref[...], w_ref[...], preferred_element_type=_f32)
        y_ref[...] = xw * dinv

    return pl.pallas_call(
        body,
        out_shape=(jax.ShapeDtypeStruct((N, 1), _f32),
                   jax.ShapeDtypeStruct((N, D), _f32)),
    )(degp, x, W1)


def _bn_relu(accp_ref, y_ref, dinv_ref, b_ref, g_ref, be_ref):
    ap = accp_ref[...]
    acc = ap[0, :N, :] + ap[1, :N, :] + y_ref[...]
    r = jnp.maximum(acc * dinv_ref[...] + b_ref[...], 0.0)
    m = jnp.mean(r, axis=0, keepdims=True)
    v = jnp.mean(r * r, axis=0, keepdims=True) - m * m
    return (r - m) * lax.rsqrt(v + 1e-5) * g_ref[...] + be_ref[...]


def _tc_layer(accp, y, dinv, b, g, be, Wn):
    def body(accp_ref, y_ref, dinv_ref, b_ref, g_ref, be_ref, w_ref, yn_ref):
        h = _bn_relu(accp_ref, y_ref, dinv_ref, b_ref, g_ref, be_ref)
        yn_ref[...] = jnp.dot(h, w_ref[...], preferred_element_type=_f32) * dinv_ref[...]

    return pl.pallas_call(
        body,
        out_shape=jax.ShapeDtypeStruct((N, D), _f32),
    )(accp, y, dinv, b, g, be, Wn)


def _tc_final(accp, y, dinv, b, g, be, batch2d, lw1, lb1, lw2, lb2):
    def body(accp_ref, y_ref, dinv_ref, b_ref, g_ref, be_ref, bat_ref,
             lw1_ref, lb1_ref, lw2_ref, lb2_ref, out_ref, pooled_ref):
        h = _bn_relu(accp_ref, y_ref, dinv_ref, b_ref, g_ref, be_ref)
        bat = bat_ref[...]
        neg = _f32(-jnp.inf)

        def pool_one(gi, carry):
            row = jnp.max(jnp.where(bat == gi, h, neg), axis=0, keepdims=True)
            pooled_ref[pl.ds(gi, 1), :] = row
            return carry

        lax.fori_loop(0, G, pool_one, 0)
        pooled = pooled_ref[...]
        h2 = jnp.maximum(
            jnp.dot(pooled, lw1_ref[...], preferred_element_type=_f32) + lb1_ref[...],
            0.0)
        out_ref[...] = jnp.dot(h2, lw2_ref[...], preferred_element_type=_f32) + lb2_ref[...]

    return pl.pallas_call(
        body,
        out_shape=jax.ShapeDtypeStruct((G, 1), _f32),
        scratch_shapes=[pltpu.VMEM((G, D), _f32)],
    )(accp, y, dinv, b, g, be, batch2d, lw1, lb1, lw2, lb2)


def kernel(x, edge_attr, edge_index, batch,
           W1, b1, g1, be1, W2, b2, g2, be2, W3, b3, g3, be3,
           lw1, lb1, lw2, lb2):
    pad = EPAD - E
    pad_dst = N + (jnp.arange(pad, dtype=jnp.int32) % (NP - N))
    srcp = jnp.concatenate(
        [edge_index[0], jnp.zeros((pad,), jnp.int32)]).reshape(NW, K, CH)
    dstp = jnp.concatenate(
        [edge_index[1], pad_dst]).reshape(NW, K, CH)
    zrows = jnp.zeros((NP, D), _f32)
    batch2d = batch.reshape(N, 1)
    b1r, g1r, be1r = b1.reshape(1, D), g1.reshape(1, D), be1.reshape(1, D)
    b2r, g2r, be2r = b2.reshape(1, D), g2.reshape(1, D), be2.reshape(1, D)
    b3r, g3r, be3r = b3.reshape(1, D), g3.reshape(1, D), be3.reshape(1, D)
    lb1r = lb1.reshape(1, D // 2)
    lb2r = lb2.reshape(1, 1)

    degp = _sc_degree(dstp, jnp.ones((CH, D), _f32), zrows)
    dinv, y1 = _tc_stage1(degp, x, W1)
    acc1 = _sc_scatter(y1, srcp, dstp, zrows)
    y2 = _tc_layer(acc1, y1, dinv, b1r, g1r, be1r, W2)
    acc2 = _sc_scatter(y2, srcp, dstp, zrows)
    y3 = _tc_layer(acc2, y2, dinv, b2r, g2r, be2r, W3)
    acc3 = _sc_scatter(y3, srcp, dstp, zrows)
    return _tc_final(acc3, y3, dinv, b3r, g3r, be3r, batch2d,
                     lw1, lb1r, lw2, lb2r)
